# baseline (speedup 1.0000x reference)
import functools

import jax
import jax.numpy as jnp
from jax import lax
from jax.experimental import pallas as pl
from jax.experimental.pallas import tpu as pltpu
from jax.experimental.pallas import tpu_sc as plsc

N = 10000
E = 320000
BLK = 128
NB = E // BLK
NBP = 2560
NCORE = 2
NSUB = 16
NW = NCORE * NSUB
ACC = 10112
ZPT = ACC // NSUB
RPT = 624
TAIL = N - NSUB * RPT
PAD_DST = N
DEGW = 16
NBUF = 4


def _core_sub():
    return lax.axis_index("c"), lax.axis_index("s")


def _zero_acc(zeros_hbm, acc, s):
    pltpu.sync_copy(zeros_hbm, acc.at[pl.ds(s * ZPT, ZPT)])


def _dump_acc(acc, out_hbm, c, s):
    r0 = s * RPT
    pltpu.sync_copy(acc.at[pl.ds(r0, RPT)], out_hbm.at[c, pl.ds(r0, RPT)])

    @pl.when(s == NSUB - 1)
    def _():
        t0 = NSUB * RPT
        pltpu.sync_copy(acc.at[pl.ds(t0, TAIL)], out_hbm.at[c, pl.ds(t0, TAIL)])


def _make_deg_kernel():
    mesh = plsc.VectorSubcoreMesh(core_axis_name="c", subcore_axis_name="s")
    bpt = NBP // NW

    @functools.partial(
        pl.kernel,
        out_type=(
            jax.ShapeDtypeStruct((NCORE, N, DEGW), jnp.float32),
            jax.ShapeDtypeStruct((NCORE, N, DEGW), jnp.float32),
        ),
        mesh=mesh,
        scratch_types=[
            pltpu.VMEM_SHARED((ACC, DEGW), jnp.float32),
            pltpu.VMEM_SHARED((ACC, DEGW), jnp.float32),
            pltpu.VMEM((bpt, BLK), jnp.int32),
            pltpu.VMEM((bpt, BLK), jnp.int32),
            pltpu.VMEM((BLK, DEGW), jnp.float32),
            pltpu.SemaphoreType.DMA,
        ],
        compiler_params=pltpu.CompilerParams(use_tc_tiling_on_sc=False),
    )
    def deg_kernel(src_hbm, dst_hbm, ones_hbm, zeros_hbm,
                   outs_hbm, outd_hbm, acc_s, acc_d, src_v, dst_v, ones_v, sem):
        c, s = _core_sub()
        w = c * NSUB + s
        pltpu.sync_copy(src_hbm.at[pl.ds(w * bpt, bpt)], src_v)
        pltpu.sync_copy(dst_hbm.at[pl.ds(w * bpt, bpt)], dst_v)
        pltpu.sync_copy(ones_hbm, ones_v)
        _zero_acc(zeros_hbm, acc_s, s)
        _zero_acc(zeros_hbm, acc_d, s)
        plsc.subcore_barrier()

        def chunk(ch, carry):
            g0 = ch * 8
            descs = []
            for j in range(8):
                descs.append(pltpu.async_copy(
                    ones_v, acc_s.at[src_v.at[g0 + j]], sem, add=True))
                descs.append(pltpu.async_copy(
                    ones_v, acc_d.at[dst_v.at[g0 + j]], sem, add=True))
            for d in descs:
                d.wait()
            return carry

        lax.fori_loop(0, bpt // 8, chunk, 0)
        plsc.subcore_barrier()
        _dump_acc(acc_s, outs_hbm, c, s)
        _dump_acc(acc_d, outd_hbm, c, s)

    return deg_kernel


def _make_agg_kernel(D):
    mesh = plsc.VectorSubcoreMesh(core_axis_name="c", subcore_axis_name="s")
    bpt = NBP // NSUB

    @functools.partial(
        pl.kernel,
        out_type=jax.ShapeDtypeStruct((NCORE, N, D), jnp.float32),
        mesh=mesh,
        scratch_types=[
            pltpu.VMEM_SHARED((ACC, D), jnp.float32),
            pltpu.VMEM((bpt, BLK), jnp.int32),
            pltpu.VMEM((bpt, BLK), jnp.int32),
            [pltpu.VMEM((BLK, D), jnp.float32) for _ in range(NBUF)],
            [pltpu.SemaphoreType.DMA for _ in range(NBUF)],
            [pltpu.SemaphoreType.DMA for _ in range(NBUF)],
        ],
        compiler_params=pltpu.CompilerParams(use_tc_tiling_on_sc=False),
    )
    def agg_kernel(y_hbm, src_hbm, dst_hbm, zeros_hbm,
                   out_hbm, acc, src_v, dst_v, rows, gsems, ssems):
        c, s = _core_sub()
        base = (c * NSUB + s) * bpt
        pltpu.sync_copy(src_hbm.at[pl.ds(base, bpt)], src_v)
        pltpu.sync_copy(dst_hbm.at[pl.ds(base, bpt)], dst_v)
        _zero_acc(zeros_hbm, acc, s)
        plsc.subcore_barrier()

        def chunk(ch, carry):
            g0 = ch * NBUF
            gds = [pltpu.async_copy(y_hbm.at[src_v.at[g0 + j]], rows[j],
                                    gsems[j]) for j in range(NBUF)]
            sds = []
            for j in range(NBUF):
                gds[j].wait()
                sds.append(pltpu.async_copy(
                    rows[j], acc.at[dst_v.at[g0 + j]], ssems[j], add=True))
            for d in sds:
                d.wait()
            return carry

        lax.fori_loop(0, bpt // NBUF, chunk, 0)
        plsc.subcore_barrier()
        _dump_acc(acc, out_hbm, c, s)

    return agg_kernel


_deg_kernel = _make_deg_kernel()
_aggq64 = _make_agg_kernel(64)
_aggq32 = _make_agg_kernel(32)


_RB = 1000
_GRID = N // _RB


def _tc0_body(hs_ref, hd_ref, x_ref, ns_ref, nd_ref, y0_ref):
    ds = jnp.sum(hs_ref[...], axis=(0, 2)) * (1.0 / DEGW)
    dd = jnp.sum(hd_ref[...], axis=(0, 2)) * (1.0 / DEGW)
    ns = lax.rsqrt(jnp.maximum(ds, 1.0))
    nd = lax.rsqrt(jnp.maximum(dd, 1.0))
    ns_ref[...] = ns[:, None]
    nd_ref[...] = nd[:, None]
    y = x_ref[...] * ns[:, None]
    y0_ref[0] = y[:, :64]
    y0_ref[1] = y[:, 64:]


def _tc0(hs, hd, x):
    return pl.pallas_call(
        _tc0_body,
        grid=(_GRID,),
        in_specs=[
            pl.BlockSpec((NCORE, _RB, DEGW), lambda i: (0, i, 0)),
            pl.BlockSpec((NCORE, _RB, DEGW), lambda i: (0, i, 0)),
            pl.BlockSpec((_RB, 128), lambda i: (i, 0)),
        ],
        out_specs=[
            pl.BlockSpec((_RB, 1), lambda i: (i, 0)),
            pl.BlockSpec((_RB, 1), lambda i: (i, 0)),
            pl.BlockSpec((NCORE, _RB, 64), lambda i: (0, i, 0)),
        ],
        out_shape=[
            jax.ShapeDtypeStruct((N, 1), jnp.float32),
            jax.ShapeDtypeStruct((N, 1), jnp.float32),
            jax.ShapeDtypeStruct((NCORE, N, 64), jnp.float32),
        ],
    )(hs, hd, x)


def _tc1_body(g0_ref, ns_ref, nd_ref, w0_ref, b0_ref, y1_ref):
    a = g0_ref[0] * nd_ref[...]
    b = g0_ref[1] * nd_ref[...]
    h = jnp.dot(a, w0_ref[0], preferred_element_type=jnp.float32)
    h = h + jnp.dot(b, w0_ref[1], preferred_element_type=jnp.float32)
    h = jnp.maximum(h + b0_ref[...], 0.0) * ns_ref[...]
    for q in range(4):
        y1_ref[q] = h[:, q * 64:(q + 1) * 64]


def _tc1(g0, ns, nd, W0, b0):
    return pl.pallas_call(
        _tc1_body,
        grid=(_GRID,),
        in_specs=[
            pl.BlockSpec((NCORE, _RB, 64), lambda i: (0, i, 0)),
            pl.BlockSpec((_RB, 1), lambda i: (i, 0)),
            pl.BlockSpec((_RB, 1), lambda i: (i, 0)),
            pl.BlockSpec((NCORE, 64, 256), lambda i: (0, 0, 0)),
            pl.BlockSpec((1, 256), lambda i: (0, 0)),
        ],
        out_specs=pl.BlockSpec((4, _RB, 64), lambda i: (0, i, 0)),
        out_shape=jax.ShapeDtypeStruct((4, N, 64), jnp.float32),
    )(g0, ns, nd, W0, b0)


def _tc2_body(g1a_ref, g1b_ref, ns_ref, nd_ref, w1_ref, b1_ref, w2_ref, y2_ref):
    nd = nd_ref[...]
    h = jnp.dot(g1a_ref[0] * nd, w1_ref[0], preferred_element_type=jnp.float32)
    h = h + jnp.dot(g1a_ref[1] * nd, w1_ref[1], preferred_element_type=jnp.float32)
    h = h + jnp.dot(g1b_ref[0] * nd, w1_ref[2], preferred_element_type=jnp.float32)
    h = h + jnp.dot(g1b_ref[1] * nd, w1_ref[3], preferred_element_type=jnp.float32)
    h = jnp.maximum(h + b1_ref[...], 0.0)
    t = jnp.dot(h, w2_ref[...], preferred_element_type=jnp.float32)
    t = t * ns_ref[...]
    y2_ref[0] = t[:, :32]
    y2_ref[1] = t[:, 32:]


def _tc2(g1a, g1b, ns, nd, W1, b1, W2):
    return pl.pallas_call(
        _tc2_body,
        grid=(_GRID,),
        in_specs=[
            pl.BlockSpec((NCORE, _RB, 64), lambda i: (0, i, 0)),
            pl.BlockSpec((NCORE, _RB, 64), lambda i: (0, i, 0)),
            pl.BlockSpec((_RB, 1), lambda i: (i, 0)),
            pl.BlockSpec((_RB, 1), lambda i: (i, 0)),
            pl.BlockSpec((4, 64, 256), lambda i: (0, 0, 0)),
            pl.BlockSpec((1, 256), lambda i: (0, 0)),
            pl.BlockSpec((256, 64), lambda i: (0, 0)),
        ],
        out_specs=pl.BlockSpec((NCORE, _RB, 32), lambda i: (0, i, 0)),
        out_shape=jax.ShapeDtypeStruct((NCORE, N, 32), jnp.float32),
    )(g1a, g1b, ns, nd, W1, b1, W2)


def _tc3_body(g2_ref, nd_ref, b2_ref, out_ref):
    t = jnp.concatenate([g2_ref[0], g2_ref[1]], axis=1)
    out_ref[...] = t * nd_ref[...] + b2_ref[...]


def _tc3(g2, nd, b2):
    return pl.pallas_call(
        _tc3_body,
        grid=(_GRID,),
        in_specs=[
            pl.BlockSpec((NCORE, _RB, 32), lambda i: (0, i, 0)),
            pl.BlockSpec((_RB, 1), lambda i: (i, 0)),
            pl.BlockSpec((1, 64), lambda i: (0, 0)),
        ],
        out_specs=pl.BlockSpec((_RB, 64), lambda i: (i, 0)),
        out_shape=jax.ShapeDtypeStruct((N, 64), jnp.float32),
    )(g2, nd, b2)


_NPADB = NBP - NB


@jax.jit
def _run(x, edge_index, W0, b0, W1, b1, W2, b2):
    src = edge_index[0].astype(jnp.int32).reshape(NB, BLK)
    dst = edge_index[1].astype(jnp.int32).reshape(NB, BLK)
    padn = jnp.full((_NPADB, BLK), PAD_DST, jnp.int32)
    pad0 = jnp.zeros((_NPADB, BLK), jnp.int32)
    src_deg = jnp.concatenate([src, padn])
    dst_pad = jnp.concatenate([dst, padn])
    src_agg = jnp.concatenate([src, pad0])
    src2 = jnp.concatenate([src_agg, src_agg + N])
    dst2 = jnp.concatenate([dst_pad, dst_pad])
    ones = jnp.ones((BLK, DEGW), jnp.float32)
    zeros_deg = jnp.zeros((ZPT, DEGW), jnp.float32)
    zeros64 = jnp.zeros((ZPT, 64), jnp.float32)
    zeros32 = jnp.zeros((ZPT, 32), jnp.float32)

    hs, hd = _deg_kernel(src_deg, dst_pad, ones, zeros_deg)
    ns, nd, y0 = _tc0(hs, hd, x)
    g0 = _aggq64(y0.reshape(2 * N, 64), src2, dst2, zeros64)
    y1 = _tc1(g0, ns, nd, W0.reshape(NCORE, 64, 256), b0.reshape(1, -1))
    g1a = _aggq64(y1[:2].reshape(2 * N, 64), src2, dst2, zeros64)
    g1b = _aggq64(y1[2:].reshape(2 * N, 64), src2, dst2, zeros64)
    y2 = _tc2(g1a, g1b, ns, nd, W1.reshape(4, 64, 256), b1.reshape(1, -1), W2)
    g2 = _aggq32(y2.reshape(2 * N, 32), src2, dst2, zeros32)
    return _tc3(g2, nd, b2.reshape(1, -1))


def kernel(x, edge_index, W0, b0, W1, b1, W2, b2):
    return _run(x, edge_index, W0, b0, W1, b1, W2, b2)
